# baseline (speedup 1.0000x reference)
import dataclasses
import functools

import jax
import jax.numpy as jnp
from jax import lax
from jax.experimental import pallas as pl
from jax.experimental.pallas import tpu as pltpu
from jax.experimental.pallas import tpu_sc as plsc

_NBINS = 65537
_BINS = 65536
_N = 1048576
_NC, _NS = 2, 16
_NW = _NC * _NS
_EPT = _N // _NS
_CHUNK = 16384
_NCHUNK = _EPT // _CHUNK
_RED_BLK = 16384
_ZDMA = 24576


def _compiler_params():
    cp = pltpu.CompilerParams(
        disable_bounds_checks=True,
        disable_semaphore_checks=True,
        skip_device_barrier=True,
    )
    if "needs_layout_passes" in pltpu.CompilerParams.__dataclass_fields__:
        cp = dataclasses.replace(cp, needs_layout_passes=False)
    return cp


def _histograms(halt_steps, rt_true):
    mesh = plsc.VectorSubcoreMesh(core_axis_name="c", subcore_axis_name="s")

    @functools.partial(
        pl.kernel,
        out_type=jax.ShapeDtypeStruct((_NW, _BINS), jnp.int32),
        mesh=mesh,
        scratch_types=[
            pltpu.VMEM((_BINS,), jnp.int32),
            pltpu.VMEM((_CHUNK,), jnp.int32),
            pltpu.VMEM((_CHUNK,), jnp.int32),
            pltpu.SemaphoreType.DMA,
            pltpu.SemaphoreType.DMA,
            pltpu.SemaphoreType.DMA,
        ],
        compiler_params=_compiler_params(),
    )
    def hist_kernel(halt_hbm, rt_hbm, zeros_hbm, out_hbm, hist, buf0, buf1,
                    sem0, sem1, sem2):
        c = lax.axis_index("c")
        s = lax.axis_index("s")
        wid = c * _NS + s
        base = s * _EPT

        zeros16 = jnp.zeros((16,), jnp.int32)
        ones16 = jnp.ones((16,), jnp.int32)

        def scatter_chunk(buf):
            v0 = buf[pl.ds(0, 16)]

            @plsc.parallel_loop(0, _CHUNK - 16, step=16, unroll=8, carry=v0)
            def vlast(g, v):
                v_next = buf[pl.ds(g + 16, 16)]
                plsc.addupdate_scatter(hist, [v], ones16)
                return v_next

            plsc.addupdate_scatter(hist, [vlast], ones16)

        def process(in_hbm):
            def start(k, buf, sem):
                pltpu.async_copy(in_hbm.at[pl.ds(base + k * _CHUNK, _CHUNK)], buf, sem)

            def wait(buf, sem):
                pltpu.make_async_copy(in_hbm.at[pl.ds(0, _CHUNK)], buf, sem).wait()

            start(0, buf0, sem0)

            zdst = hist.at[pl.ds(_BINS - _ZDMA, _ZDMA)]
            pltpu.async_copy(zeros_hbm.at[s], zdst, sem2)

            @plsc.parallel_loop(0, _BINS - _ZDMA, step=16, unroll=8)
            def _(i):
                hist[pl.ds(i, 16)] = zeros16

            pltpu.make_async_copy(zeros_hbm.at[s], zdst, sem2).wait()

            @pl.loop(0, _NCHUNK, step=2)
            def _(k):
                wait(buf0, sem0)
                start(k + 1, buf1, sem1)
                scatter_chunk(buf0)
                wait(buf1, sem1)

                @pl.when(k + 2 < _NCHUNK)
                def _():
                    start(k + 2, buf0, sem0)

                scatter_chunk(buf1)

        @pl.when(c == 0)
        def _():
            process(halt_hbm)

        @pl.when(c == 1)
        def _():
            process(rt_hbm)

        pltpu.sync_copy(hist, out_hbm.at[wid])

    zeros_rows = jnp.zeros((_NS, _ZDMA), jnp.int32)
    return hist_kernel(halt_steps, rt_true, zeros_rows)


def _reduce_body(parts_ref, out_ref, acc_ref):
    i = pl.program_id(0)
    parts = parts_ref[...]
    p = jnp.sum(parts[0:_NS], axis=0).astype(jnp.float32)
    t = jnp.sum(parts[_NS:_NW], axis=0).astype(jnp.float32)
    part = jnp.sum(jnp.exp(t) * (t - p))

    @pl.when(i == 0)
    def _():
        acc_ref[0] = part

    @pl.when(i > 0)
    def _():
        acc_ref[0] += part

    @pl.when(i == pl.num_programs(0) - 1)
    def _():
        out_ref[...] = (acc_ref[0] * (1.0 / float(_NBINS))).reshape(1, 1)


def kernel(halt_steps, rt_true):
    parts = _histograms(halt_steps, rt_true)
    loss = pl.pallas_call(
        _reduce_body,
        grid=(_BINS // _RED_BLK,),
        in_specs=[pl.BlockSpec((_NW, _RED_BLK), lambda i: (0, i))],
        out_specs=pl.BlockSpec((1, 1), lambda i: (0, 0)),
        out_shape=jax.ShapeDtypeStruct((1, 1), jnp.float32),
        scratch_shapes=[pltpu.SMEM((1,), jnp.float32)],
        compiler_params=pltpu.CompilerParams(
            disable_bounds_checks=True,
            skip_device_barrier=True,
            dimension_semantics=("arbitrary",),
        ),
    )(parts)
    return loss[0, 0]


# speedup vs baseline: 1.0323x; 1.0323x over previous
import dataclasses
import functools

import jax
import jax.numpy as jnp
from jax import lax
from jax.experimental import pallas as pl
from jax.experimental.pallas import tpu as pltpu
from jax.experimental.pallas import tpu_sc as plsc

_NBINS = 65537
_BINS = 65536
_N = 1048576
_NC, _NS = 2, 16
_NW = _NC * _NS
_EPT = _N // _NS
_CHUNK = 16384
_NCHUNK = _EPT // _CHUNK
_RED_BLK = 32768


def _compiler_params():
    cp = pltpu.CompilerParams(
        disable_bounds_checks=True,
        disable_semaphore_checks=True,
        skip_device_barrier=True,
    )
    if "needs_layout_passes" in pltpu.CompilerParams.__dataclass_fields__:
        cp = dataclasses.replace(cp, needs_layout_passes=False)
    return cp


def _histograms(halt_steps, rt_true):
    mesh = plsc.VectorSubcoreMesh(core_axis_name="c", subcore_axis_name="s")

    @functools.partial(
        pl.kernel,
        out_type=jax.ShapeDtypeStruct((_NW, _BINS), jnp.int32),
        mesh=mesh,
        scratch_types=[
            pltpu.VMEM((_BINS,), jnp.int32),
            pltpu.VMEM((_CHUNK,), jnp.int32),
            pltpu.VMEM((_CHUNK,), jnp.int32),
            pltpu.SemaphoreType.DMA,
            pltpu.SemaphoreType.DMA,
        ],
        compiler_params=_compiler_params(),
    )
    def hist_kernel(halt_hbm, rt_hbm, out_hbm, hist, buf0, buf1, sem0, sem1):
        c = lax.axis_index("c")
        s = lax.axis_index("s")
        wid = c * _NS + s
        base = s * _EPT

        zeros16 = jnp.zeros((16,), jnp.int32)
        ones16 = jnp.ones((16,), jnp.int32)

        def scatter_chunk(buf):
            v0 = buf[pl.ds(0, 16)]

            @plsc.parallel_loop(0, _CHUNK - 16, step=16, unroll=8, carry=v0)
            def vlast(g, v):
                v_next = buf[pl.ds(g + 16, 16)]
                plsc.addupdate_scatter(hist, [v], ones16)
                return v_next

            plsc.addupdate_scatter(hist, [vlast], ones16)

        def process(in_hbm):
            def start(k, buf, sem):
                pltpu.async_copy(in_hbm.at[pl.ds(base + k * _CHUNK, _CHUNK)], buf, sem)

            def wait(buf, sem):
                pltpu.make_async_copy(in_hbm.at[pl.ds(0, _CHUNK)], buf, sem).wait()

            start(0, buf0, sem0)

            @plsc.parallel_loop(0, _BINS, step=16, unroll=8)
            def _(i):
                hist[pl.ds(i, 16)] = zeros16

            @pl.loop(0, _NCHUNK, step=2)
            def _(k):
                wait(buf0, sem0)
                start(k + 1, buf1, sem1)
                scatter_chunk(buf0)
                wait(buf1, sem1)

                @pl.when(k + 2 < _NCHUNK)
                def _():
                    start(k + 2, buf0, sem0)

                scatter_chunk(buf1)

        @pl.when(c == 0)
        def _():
            process(halt_hbm)

        @pl.when(c == 1)
        def _():
            process(rt_hbm)

        pltpu.sync_copy(hist, out_hbm.at[wid])

    return hist_kernel(halt_steps, rt_true)


def _reduce_body(parts_ref, out_ref, acc_ref):
    i = pl.program_id(0)
    parts = parts_ref[...]
    p = jnp.sum(parts[0:_NS], axis=0).astype(jnp.float32)
    t = jnp.sum(parts[_NS:_NW], axis=0).astype(jnp.float32)
    part = jnp.sum(jnp.exp(t) * (t - p))

    @pl.when(i == 0)
    def _():
        acc_ref[0] = part

    @pl.when(i > 0)
    def _():
        acc_ref[0] += part

    @pl.when(i == pl.num_programs(0) - 1)
    def _():
        out_ref[...] = (acc_ref[0] * (1.0 / float(_NBINS))).reshape(1, 1)


def kernel(halt_steps, rt_true):
    parts = _histograms(halt_steps, rt_true)
    loss = pl.pallas_call(
        _reduce_body,
        grid=(_BINS // _RED_BLK,),
        in_specs=[pl.BlockSpec((_NW, _RED_BLK), lambda i: (0, i))],
        out_specs=pl.BlockSpec((1, 1), lambda i: (0, 0)),
        out_shape=jax.ShapeDtypeStruct((1, 1), jnp.float32),
        scratch_shapes=[pltpu.SMEM((1,), jnp.float32)],
        compiler_params=pltpu.CompilerParams(
            disable_bounds_checks=True,
            skip_device_barrier=True,
            dimension_semantics=("arbitrary",),
        ),
    )(parts)
    return loss[0, 0]
